# baseline (speedup 1.0000x reference)
import functools

import jax
import jax.numpy as jnp
from jax.experimental import pallas as pl
from jax.experimental.pallas import tpu as pltpu


def _moe_body(x_ref, w1_ref, w2_ref, rw_ref, o_ref, acc_ref, *, d, n_e):
    e = pl.program_id(1)
    x = x_ref[...]
    gu = jnp.dot(x, w1_ref[0], preferred_element_type=jnp.float32)
    gate = gu[:, :d]
    up = gu[:, d:]
    rw_all = rw_ref[...]
    mask = jax.lax.broadcasted_iota(jnp.int32, rw_all.shape, 1) == e
    rw = jnp.sum(jnp.where(mask, rw_all, 0.0), axis=1, keepdims=True)
    g = up * (gate * jax.nn.sigmoid(gate)) * rw
    y = jnp.dot(g.astype(jnp.bfloat16), w2_ref[0],
                preferred_element_type=jnp.float32)

    @pl.when(e == 0)
    def _():
        acc_ref[...] = y

    @pl.when(e != 0)
    def _():
        acc_ref[...] += y

    @pl.when(e == n_e - 1)
    def _():
        o_ref[...] = acc_ref[...].astype(jnp.bfloat16)


def kernel(hidden_states, routing_weights, router_indices, gate_up_proj, down_proj):
    del router_indices
    t, h = hidden_states.shape
    n_e = routing_weights.shape[1]
    d2 = gate_up_proj.shape[1]
    d = d2 // 2
    w1 = gate_up_proj.reshape(n_e, h, d2).astype(jnp.bfloat16)
    w2 = down_proj.reshape(n_e, d, h).astype(jnp.bfloat16)
    x16 = hidden_states.astype(jnp.bfloat16)
    tb = min(t, 1024)
    out = pl.pallas_call(
        functools.partial(_moe_body, d=d, n_e=n_e),
        grid=(t // tb, n_e),
        in_specs=[
            pl.BlockSpec((tb, h), lambda i, e: (i, 0)),
            pl.BlockSpec((1, h, d2), lambda i, e: (e, 0, 0)),
            pl.BlockSpec((1, d, h), lambda i, e: (e, 0, 0)),
            pl.BlockSpec((tb, n_e), lambda i, e: (i, 0)),
        ],
        out_specs=pl.BlockSpec((tb, h), lambda i, e: (i, 0)),
        out_shape=jax.ShapeDtypeStruct((t, h), jnp.bfloat16),
        scratch_shapes=[pltpu.VMEM((tb, h), jnp.float32)],
        compiler_params=pltpu.CompilerParams(
            dimension_semantics=("parallel", "arbitrary"),
            vmem_limit_bytes=64 * 1024 * 1024,
        ),
    )(x16, w1, w2, routing_weights)
    return out.astype(jnp.float32).reshape(t, 1, h)


# speedup vs baseline: 1.1286x; 1.1286x over previous
import functools

import jax
import jax.numpy as jnp
from jax.experimental import pallas as pl
from jax.experimental.pallas import tpu as pltpu


def _moe_body(x_ref, w1_ref, w2_ref, rw_ref, o_ref, g_ref, *, d, n_e):
    e = pl.program_id(1)
    x = x_ref[...]
    gu = jnp.dot(x, w1_ref[0], preferred_element_type=jnp.float32)
    gate = gu[:, :d]
    up = gu[:, d:]
    rw_all = rw_ref[...]
    mask = jax.lax.broadcasted_iota(jnp.int32, rw_all.shape, 1) == e
    rw = jnp.sum(jnp.where(mask, rw_all, 0.0), axis=1, keepdims=True)
    g = up * (gate * jax.nn.sigmoid(gate)) * rw
    g_ref[e] = g.astype(jnp.bfloat16)

    @pl.when(e == n_e - 1)
    def _():
        acc = jnp.dot(g_ref[0], w2_ref[0], preferred_element_type=jnp.float32)
        for ee in range(1, n_e):
            acc += jnp.dot(g_ref[ee], w2_ref[ee],
                           preferred_element_type=jnp.float32)
        o_ref[:, 0, :] = acc


def kernel(hidden_states, routing_weights, router_indices, gate_up_proj, down_proj):
    del router_indices
    t, h = hidden_states.shape
    n_e = routing_weights.shape[1]
    d2 = gate_up_proj.shape[1]
    d = d2 // 2
    w1 = gate_up_proj.reshape(n_e, h, d2).astype(jnp.bfloat16)
    w2 = down_proj.reshape(n_e, d, h).astype(jnp.bfloat16)
    x16 = hidden_states.astype(jnp.bfloat16)
    tb = min(t, 512)
    out = pl.pallas_call(
        functools.partial(_moe_body, d=d, n_e=n_e),
        grid=(t // tb, n_e),
        in_specs=[
            pl.BlockSpec((tb, h), lambda i, e: (i, 0)),
            pl.BlockSpec((1, h, d2), lambda i, e: (e, 0, 0)),
            pl.BlockSpec((n_e, d, h), lambda i, e: (0, 0, 0)),
            pl.BlockSpec((tb, n_e), lambda i, e: (i, 0)),
        ],
        out_specs=pl.BlockSpec((tb, 1, h), lambda i, e: (i, 0, 0)),
        out_shape=jax.ShapeDtypeStruct((t, 1, h), jnp.float32),
        scratch_shapes=[pltpu.VMEM((n_e, tb, d), jnp.bfloat16)],
        compiler_params=pltpu.CompilerParams(
            dimension_semantics=("parallel", "arbitrary"),
            vmem_limit_bytes=64 * 1024 * 1024,
        ),
    )(x16, w1, w2, routing_weights)
    return out


# speedup vs baseline: 1.1757x; 1.0417x over previous
import functools

import jax
import jax.numpy as jnp
from jax.experimental import pallas as pl
from jax.experimental.pallas import tpu as pltpu


def _moe_body(x_ref, w1_ref, w2_ref, rw_ref, o_ref, *, d, n_e):
    e = pl.program_id(1)
    x = x_ref[...]
    gu = jnp.dot(x, w1_ref[0], preferred_element_type=jnp.float32)
    gate = gu[:, :d]
    up = gu[:, d:]
    rw_all = rw_ref[...]
    mask = jax.lax.broadcasted_iota(jnp.int32, rw_all.shape, 1) == e
    rw = jnp.sum(jnp.where(mask, rw_all, 0.0), axis=1, keepdims=True)
    g = up * (gate * jax.nn.sigmoid(gate)) * rw
    y = jnp.dot(g.astype(jnp.bfloat16), w2_ref[0],
                preferred_element_type=jnp.float32)

    @pl.when(e == 0)
    def _():
        o_ref[:, 0, :] = y

    @pl.when(e != 0)
    def _():
        o_ref[:, 0, :] += y


def kernel(hidden_states, routing_weights, router_indices, gate_up_proj, down_proj):
    del router_indices
    t, h = hidden_states.shape
    n_e = routing_weights.shape[1]
    d2 = gate_up_proj.shape[1]
    d = d2 // 2
    w1 = gate_up_proj.reshape(n_e, h, d2)
    w2 = down_proj.reshape(n_e, d, h).astype(jnp.bfloat16)
    tb = min(t, 512)
    out = pl.pallas_call(
        functools.partial(_moe_body, d=d, n_e=n_e),
        grid=(t // tb, n_e),
        in_specs=[
            pl.BlockSpec((tb, h), lambda i, e: (i, 0)),
            pl.BlockSpec((1, h, d2), lambda i, e: (e, 0, 0)),
            pl.BlockSpec((1, d, h), lambda i, e: (e, 0, 0)),
            pl.BlockSpec((tb, n_e), lambda i, e: (i, 0)),
        ],
        out_specs=pl.BlockSpec((tb, 1, h), lambda i, e: (i, 0, 0)),
        out_shape=jax.ShapeDtypeStruct((t, 1, h), jnp.float32),
        compiler_params=pltpu.CompilerParams(
            dimension_semantics=("parallel", "arbitrary"),
            vmem_limit_bytes=64 * 1024 * 1024,
        ),
    )(hidden_states, w1, w2, routing_weights)
    return out
